# baseline (speedup 1.0000x reference)
import functools

import jax
import jax.numpy as jnp
from jax.experimental import pallas as pl
from jax.experimental.pallas import tpu as pltpu

D_MODEL = 768
FF_DIM = 512
N_EXPERTS = 8
TOP_K = 2
T_BLK = 256


def _moe_block_kernel(x_ref, wr_ref, w1_ref, w2_ref, out_ref, rw_ref):
    e = pl.program_id(1)

    @pl.when(e == 0)
    def _compute_router():
        x = x_ref[...]
        logits = jax.lax.dot_general(
            x, wr_ref[...], (((1,), (1,)), ((), ())),
            preferred_element_type=jnp.float32)
        col = jax.lax.broadcasted_iota(jnp.int32, logits.shape, 1)
        m0 = jnp.max(logits, axis=1, keepdims=True)
        is0 = logits == m0
        i0 = jnp.min(jnp.where(is0, col, N_EXPERTS), axis=1, keepdims=True)
        oh0 = col == i0
        masked = jnp.where(oh0, -jnp.inf, logits)
        m1 = jnp.max(masked, axis=1, keepdims=True)
        is1 = masked == m1
        i1 = jnp.min(jnp.where(is1, col, N_EXPERTS), axis=1, keepdims=True)
        oh1 = col == i1
        w0 = 1.0 / (1.0 + jnp.exp(m1 - m0))
        rw_ref[...] = jnp.where(oh0, w0, 0.0) + jnp.where(oh1, 1.0 - w0, 0.0)

    x = x_ref[...]
    h = jax.lax.dot_general(
        x, w1_ref[0], (((1,), (1,)), ((), ())),
        preferred_element_type=jnp.float32)
    xpart = h[:, :FF_DIM]
    gate = h[:, FF_DIM:]
    act = xpart * (gate * jax.nn.sigmoid(gate))
    o = jax.lax.dot_general(
        act, w2_ref[0], (((1,), (1,)), ((), ())),
        preferred_element_type=jnp.float32)
    rw = rw_ref[...]
    ecol = jax.lax.broadcasted_iota(jnp.int32, rw.shape, 1)
    rw_e = jnp.sum(jnp.where(ecol == e, rw, 0.0), axis=1, keepdims=True)
    scaled = rw_e * o

    @pl.when(e == 0)
    def _init():
        out_ref[...] = scaled

    @pl.when(e != 0)
    def _acc():
        out_ref[...] += scaled


@functools.partial(jax.jit, static_argnames=())
def kernel(x, W_router, W1, W2):
    B, T, C = x.shape
    flat = x.reshape(-1, C)
    n_tok = flat.shape[0]
    grid = (n_tok // T_BLK, N_EXPERTS)
    out = pl.pallas_call(
        _moe_block_kernel,
        grid=grid,
        in_specs=[
            pl.BlockSpec((T_BLK, C), lambda t, e: (t, 0)),
            pl.BlockSpec((N_EXPERTS, C), lambda t, e: (0, 0)),
            pl.BlockSpec((1, 2 * FF_DIM, C), lambda t, e: (e, 0, 0)),
            pl.BlockSpec((1, C, FF_DIM), lambda t, e: (e, 0, 0)),
        ],
        out_specs=pl.BlockSpec((T_BLK, C), lambda t, e: (t, 0)),
        out_shape=jax.ShapeDtypeStruct((n_tok, C), jnp.float32),
        scratch_shapes=[pltpu.VMEM((T_BLK, N_EXPERTS), jnp.float32)],
        compiler_params=pltpu.CompilerParams(
            dimension_semantics=("arbitrary", "arbitrary"),
        ),
    )(flat, W_router, W1, W2)
    return out.reshape(B, T, C)


# speedup vs baseline: 1.4938x; 1.4938x over previous
import functools

import jax
import jax.numpy as jnp
from jax.experimental import pallas as pl
from jax.experimental.pallas import tpu as pltpu

D_MODEL = 768
FF_DIM = 512
N_EXPERTS = 8
TOP_K = 2
T_BLK = 1024


def _moe_block_kernel(x_ref, xb_ref, wr_ref, w1_ref, w2_ref, out_ref, rw_ref):
    e = pl.program_id(1)

    @pl.when(e == 0)
    def _compute_router():
        x = x_ref[...]
        logits = jax.lax.dot_general(
            x, wr_ref[...], (((1,), (1,)), ((), ())),
            preferred_element_type=jnp.float32)
        col = jax.lax.broadcasted_iota(jnp.int32, logits.shape, 1)
        m0 = jnp.max(logits, axis=1, keepdims=True)
        is0 = logits == m0
        i0 = jnp.min(jnp.where(is0, col, N_EXPERTS), axis=1, keepdims=True)
        oh0 = col == i0
        masked = jnp.where(oh0, -jnp.inf, logits)
        m1 = jnp.max(masked, axis=1, keepdims=True)
        is1 = masked == m1
        i1 = jnp.min(jnp.where(is1, col, N_EXPERTS), axis=1, keepdims=True)
        oh1 = col == i1
        w0 = 1.0 / (1.0 + jnp.exp(m1 - m0))
        rw_ref[...] = jnp.where(oh0, w0, 0.0) + jnp.where(oh1, 1.0 - w0, 0.0)

    xb = xb_ref[...]
    h = jax.lax.dot_general(
        xb, w1_ref[0], (((1,), (1,)), ((), ())),
        preferred_element_type=jnp.float32)
    xpart = h[:, :FF_DIM]
    gate = h[:, FF_DIM:]
    act = (xpart * (gate * jax.nn.sigmoid(gate))).astype(jnp.bfloat16)
    o = jax.lax.dot_general(
        act, w2_ref[0], (((1,), (1,)), ((), ())),
        preferred_element_type=jnp.float32)
    rw = rw_ref[...]
    ecol = jax.lax.broadcasted_iota(jnp.int32, rw.shape, 1)
    rw_e = jnp.sum(jnp.where(ecol == e, rw, 0.0), axis=1, keepdims=True)
    scaled = rw_e * o

    @pl.when(e == 0)
    def _init():
        out_ref[...] = scaled

    @pl.when(e != 0)
    def _acc():
        out_ref[...] += scaled


@functools.partial(jax.jit, static_argnames=())
def kernel(x, W_router, W1, W2):
    B, T, C = x.shape
    flat = x.reshape(-1, C)
    flat_bf = flat.astype(jnp.bfloat16)
    w1_bf = W1.astype(jnp.bfloat16)
    w2_bf = W2.astype(jnp.bfloat16)
    n_tok = flat.shape[0]
    grid = (n_tok // T_BLK, N_EXPERTS)
    out = pl.pallas_call(
        _moe_block_kernel,
        grid=grid,
        in_specs=[
            pl.BlockSpec((T_BLK, C), lambda t, e: (t, 0)),
            pl.BlockSpec((T_BLK, C), lambda t, e: (t, 0)),
            pl.BlockSpec((N_EXPERTS, C), lambda t, e: (0, 0)),
            pl.BlockSpec((1, 2 * FF_DIM, C), lambda t, e: (e, 0, 0)),
            pl.BlockSpec((1, C, FF_DIM), lambda t, e: (e, 0, 0)),
        ],
        out_specs=pl.BlockSpec((T_BLK, C), lambda t, e: (t, 0)),
        out_shape=jax.ShapeDtypeStruct((n_tok, C), jnp.float32),
        scratch_shapes=[pltpu.VMEM((T_BLK, N_EXPERTS), jnp.float32)],
        compiler_params=pltpu.CompilerParams(
            dimension_semantics=("arbitrary", "arbitrary"),
        ),
    )(flat, flat_bf, W_router, w1_bf, w2_bf)
    return out.reshape(B, T, C)


# speedup vs baseline: 2.1442x; 1.4354x over previous
import functools

import jax
import jax.numpy as jnp
from jax.experimental import pallas as pl
from jax.experimental.pallas import tpu as pltpu

D_MODEL = 768
FF_DIM = 512
N_EXPERTS = 8
TOP_K = 2
T_BLK = 2048


def _moe_block_kernel(x_ref, wr_ref, w1_ref, w2_ref, out_ref, rw_ref, xb_ref):
    e = pl.program_id(1)

    @pl.when(e == 0)
    def _compute_router():
        x = x_ref[...]
        xb_ref[...] = x.astype(jnp.bfloat16)
        logits = jax.lax.dot_general(
            x, wr_ref[...], (((1,), (1,)), ((), ())),
            preferred_element_type=jnp.float32)
        col = jax.lax.broadcasted_iota(jnp.int32, logits.shape, 1)
        m0 = jnp.max(logits, axis=1, keepdims=True)
        is0 = logits == m0
        i0 = jnp.min(jnp.where(is0, col, N_EXPERTS), axis=1, keepdims=True)
        oh0 = col == i0
        masked = jnp.where(oh0, -jnp.inf, logits)
        m1 = jnp.max(masked, axis=1, keepdims=True)
        is1 = masked == m1
        i1 = jnp.min(jnp.where(is1, col, N_EXPERTS), axis=1, keepdims=True)
        oh1 = col == i1
        w0 = 1.0 / (1.0 + jnp.exp(m1 - m0))
        rw_ref[...] = jnp.where(oh0, w0, 0.0) + jnp.where(oh1, 1.0 - w0, 0.0)

    xb = xb_ref[...]
    w1b = w1_ref[0].astype(jnp.bfloat16)
    h = jax.lax.dot_general(
        xb, w1b, (((1,), (1,)), ((), ())),
        preferred_element_type=jnp.float32)
    xpart = h[:, :FF_DIM]
    gate = h[:, FF_DIM:]
    act = (xpart * (gate * jax.nn.sigmoid(gate))).astype(jnp.bfloat16)
    w2b = w2_ref[0].astype(jnp.bfloat16)
    o = jax.lax.dot_general(
        act, w2b, (((1,), (1,)), ((), ())),
        preferred_element_type=jnp.float32)
    rw = rw_ref[...]
    ecol = jax.lax.broadcasted_iota(jnp.int32, rw.shape, 1)
    rw_e = jnp.sum(jnp.where(ecol == e, rw, 0.0), axis=1, keepdims=True)
    scaled = rw_e * o

    @pl.when(e == 0)
    def _init():
        out_ref[...] = scaled

    @pl.when(e != 0)
    def _acc():
        out_ref[...] += scaled


@functools.partial(jax.jit, static_argnames=())
def kernel(x, W_router, W1, W2):
    B, T, C = x.shape
    flat = x.reshape(-1, C)
    n_tok = flat.shape[0]
    grid = (n_tok // T_BLK, N_EXPERTS)
    out = pl.pallas_call(
        _moe_block_kernel,
        grid=grid,
        in_specs=[
            pl.BlockSpec((T_BLK, C), lambda t, e: (t, 0)),
            pl.BlockSpec((N_EXPERTS, C), lambda t, e: (0, 0)),
            pl.BlockSpec((1, 2 * FF_DIM, C), lambda t, e: (e, 0, 0)),
            pl.BlockSpec((1, C, FF_DIM), lambda t, e: (e, 0, 0)),
        ],
        out_specs=pl.BlockSpec((T_BLK, C), lambda t, e: (t, 0)),
        out_shape=jax.ShapeDtypeStruct((n_tok, C), jnp.float32),
        scratch_shapes=[
            pltpu.VMEM((T_BLK, N_EXPERTS), jnp.float32),
            pltpu.VMEM((T_BLK, C), jnp.bfloat16),
        ],
        compiler_params=pltpu.CompilerParams(
            dimension_semantics=("arbitrary", "arbitrary"),
        ),
    )(flat, W_router, W1, W2)
    return out.reshape(B, T, C)
